# Initial kernel scaffold; baseline (speedup 1.0000x reference)
import jax
import jax.numpy as jnp
from jax.experimental import pallas as pl


def kernel(x_seq, edge_index, edge_weight, Wruf0, Wrub0, bru0, Wcf0, Wcb0, bc0, Wruf1, Wrub1, bru1, Wcf1, Wcb1, bc1):
    raise NotImplementedError("write your pallas kernel here")


# speedup vs baseline: 5.1336x; 5.1336x over previous
import jax
import jax.numpy as jnp
from jax import lax
from jax.experimental import pallas as pl
from jax.experimental.pallas import tpu as pltpu
from jax.experimental.pallas import tpu_sc as plsc

N = 10000
E = 160000
IN_CH = 128
HID = 64
NLAYERS = 2
BATCH = 2
TSTEPS = 6

NC = 2
NS = 16
CH = 128
CW = 128
EPT = E // NS
NCHUNK = 80
EPTP = NCHUNK * CH
NPAD = 10240
NROUNDS = 4
NQR = NPAD // NROUNDS
RPQ = NQR // NS

_mesh = plsc.VectorSubcoreMesh(
    core_axis_name="c", subcore_axis_name="s", num_cores=NC, num_subcores=NS)


def _make_dconv():
    def body(gidx, sidx, wts, cnts, degs, p2, p1, p0, y1, out,
             gidx_t, sidx_t, wt_t, cnt_t, gb0, gb1, degbuf, acc, sem0, sem1):
        cid = lax.axis_index("c")
        sid = lax.axis_index("s")
        cc = cid
        iota16 = lax.iota(jnp.int32, 16)

        for d in range(2):
            fl = pl.ds(d * EPTP, EPTP)
            pltpu.sync_copy(gidx.at[d, sid], gidx_t.at[fl])
            pltpu.sync_copy(sidx.at[d, sid], sidx_t.at[fl])
            pltpu.sync_copy(wts.at[d, sid], wt_t.at[fl])
            pltpu.sync_copy(cnts.at[d, sid], cnt_t.at[pl.ds(d * 128, 128)])
        cv = [cnt_t[pl.ds(0, 16)], cnt_t[pl.ds(128, 16)]]
        cb = [[0, cv[d][0], cv[d][1], cv[d][2], EPTP] for d in range(2)]

        def scale_scatter(buf, d, r, lo, hi):
            def sb(g, c):
                eb = d * EPTP + r * CH + g * 16
                wvec = wt_t[pl.ds(eb, 16)]
                pos = iota16 + (r * CH + g * 16)
                m = (pos >= lo) & (pos < hi)
                wvec = jnp.where(m, wvec, 0.0)
                for lane in range(16):
                    w = wvec[lane]
                    e = g * 16 + lane
                    for kk in range(CW // 16):
                        sl = pl.ds(kk * 16, 16)
                        buf[e, sl] = buf[e, sl] * w
                iv = sidx_t[pl.ds(eb, 16)]
                pltpu.sync_copy(buf.at[pl.ds(g * 16, 16)], acc.at[iv],
                                add=True)
                return c
            lax.fori_loop(0, CH // 16, sb, 0)

        def sweep(table, d, h):
            lo = cb[d][h]
            hi = cb[d][h + 1]
            t0 = lo // CH
            tend = (hi + CH - 1) // CH
            trips = tend - t0

            def gref(r):
                return table.at[gidx_t.at[pl.ds(d * EPTP + r * CH, CH)]]

            def wait(sem, buf):
                pltpu.make_async_copy(table.at[pl.ds(0, CH)], buf, sem).wait()

            @pl.when(trips > 0)
            def _():
                pltpu.async_copy(gref(t0), gb0, sem0)

            def lbody(jj, c):
                r0 = t0 + 2 * jj
                r1 = r0 + 1
                wait(sem0, gb0)

                @pl.when(r1 < tend)
                def _():
                    pltpu.async_copy(gref(r1), gb1, sem1)

                scale_scatter(gb0, d, r0, lo, hi)

                @pl.when(r1 < tend)
                def _():
                    wait(sem1, gb1)

                    @pl.when(r1 + 1 < tend)
                    def _():
                        pltpu.async_copy(gref(r1 + 1), gb0, sem0)

                    scale_scatter(gb1, d, r1, lo, hi)
                return c

            lax.fori_loop(0, (trips + 1) // 2, lbody, 0)

        def init_round(src_tab, h):
            pltpu.sync_copy(src_tab.at[pl.ds(h * NQR + sid * RPQ, RPQ)],
                            acc.at[pl.ds(sid * RPQ, RPQ)])
            plsc.subcore_barrier()

        def writeout(d, h, dst, scaled):
            if scaled:
                pltpu.sync_copy(
                    degs.at[d, pl.ds((h * NQR + sid * RPQ) * 16, RPQ * 16)],
                    degbuf)
            for q in range(RPQ // 80):
                qa = sid * RPQ + q * 80
                qd = h * NQR + qa
                pltpu.sync_copy(acc.at[pl.ds(qa, 80)], gb0.at[pl.ds(0, 80)])
                if scaled:
                    def srow(r, c):
                        dvec = degbuf[pl.ds((q * 80 + r) * 16, 16)]
                        inv = 1.0 / jnp.maximum(dvec, 1e-10)
                        for kk in range(CW // 16):
                            sl = pl.ds(kk * 16, 16)
                            gb0[r, sl] = gb0[r, sl] * inv
                        return c
                    lax.fori_loop(0, 80, srow, 0)
                pltpu.sync_copy(gb0.at[pl.ds(0, 80)], dst.at[pl.ds(qd, 80)])

        for d in range(2):
            for h in range(NROUNDS):
                init_round(p1.at[d, cc], h)
                sweep(p2.at[d, cc], d, h)
                plsc.subcore_barrier()
                writeout(d, h, y1.at[d, cc], scaled=True)
        plsc.subcore_barrier()
        for h in range(NROUNDS):
            init_round(p0.at[cc], h)
            for d in range(2):
                sweep(y1.at[d, cc], d, h)
            plsc.subcore_barrier()
            writeout(0, h, out.at[cc], scaled=False)

    return pl.kernel(
        body,
        out_type=[jax.ShapeDtypeStruct((2, BATCH, NPAD, CW), jnp.float32),
                  jax.ShapeDtypeStruct((BATCH, NPAD, CW), jnp.float32)],
        mesh=_mesh,
        scratch_types=[
            pltpu.VMEM((2 * EPTP,), jnp.int32),
            pltpu.VMEM((2 * EPTP,), jnp.int32),
            pltpu.VMEM((2 * EPTP,), jnp.float32),
            pltpu.VMEM((256,), jnp.int32),
            pltpu.VMEM((CH, CW), jnp.float32),
            pltpu.VMEM((CH, CW), jnp.float32),
            pltpu.VMEM((RPQ * 16,), jnp.float32),
            pltpu.VMEM_SHARED((NQR, CW), jnp.float32),
            pltpu.SemaphoreType.DMA,
            pltpu.SemaphoreType.DMA,
        ],
    )


_dconv = _make_dconv()


_BLK = 2000
_NB = N // _BLK


def _dinv(deg_blk):
    return 1.0 / jnp.maximum(deg_blk[:, 0:1], 1e-10)


def _make_ru_pre(d_in):
    ow = 2 * HID
    d = d_in + HID

    def body(x_ref, h_ref, w_ref, b_ref, degf_ref, degb_ref,
             p2_ref, p1_ref, p0_ref):
        xh = jnp.concatenate([x_ref[0], h_ref[0]], axis=-1)
        res = jnp.dot(xh, w_ref[...], preferred_element_type=jnp.float32)
        df = _dinv(degf_ref[...])
        db = _dinv(degb_ref[...])
        p2_ref[0, 0] = res[:, 0 * ow:1 * ow] * df
        p1_ref[0, 0] = res[:, 1 * ow:2 * ow]
        p2_ref[1, 0] = res[:, 3 * ow:4 * ow] * db
        p1_ref[1, 0] = res[:, 4 * ow:5 * ow]
        p0_ref[0] = res[:, 2 * ow:3 * ow] + res[:, 5 * ow:6 * ow] + b_ref[...]

    return pl.pallas_call(
        body,
        grid=(BATCH, _NB),
        in_specs=[
            pl.BlockSpec((1, _BLK, d_in), lambda b, i: (b, i, 0)),
            pl.BlockSpec((1, _BLK, HID), lambda b, i: (b, i, 0)),
            pl.BlockSpec((d, 6 * ow), lambda b, i: (0, 0)),
            pl.BlockSpec((1, ow), lambda b, i: (0, 0)),
            pl.BlockSpec((_BLK, 16), lambda b, i: (i, 0)),
            pl.BlockSpec((_BLK, 16), lambda b, i: (i, 0)),
        ],
        out_specs=[
            pl.BlockSpec((2, 1, _BLK, CW), lambda b, i: (0, b, i, 0)),
            pl.BlockSpec((2, 1, _BLK, CW), lambda b, i: (0, b, i, 0)),
            pl.BlockSpec((1, _BLK, CW), lambda b, i: (b, i, 0)),
        ],
        out_shape=[
            jax.ShapeDtypeStruct((2, BATCH, NPAD, CW), jnp.float32),
            jax.ShapeDtypeStruct((2, BATCH, NPAD, CW), jnp.float32),
            jax.ShapeDtypeStruct((BATCH, NPAD, CW), jnp.float32),
        ],
    )


def _make_c_pre(d_in):
    ow = HID
    d = d_in + HID

    def body(ru_ref, x_ref, h_ref, w_ref, b_ref, degf_ref, degb_ref,
             p2_ref, p1_ref, p0_ref, u_ref):
        ruv = jax.nn.sigmoid(ru_ref[0])
        r = ruv[:, :HID]
        u = ruv[:, HID:]
        rh = r * h_ref[0]
        xc = jnp.concatenate([x_ref[0], rh], axis=-1)
        res = jnp.dot(xc, w_ref[...], preferred_element_type=jnp.float32)
        df = _dinv(degf_ref[...])
        db = _dinv(degb_ref[...])
        zpad = jnp.zeros((x_ref.shape[1], CW - ow), jnp.float32)
        p0full = res[:, 2 * ow:3 * ow] + res[:, 5 * ow:6 * ow] + b_ref[...]
        p2_ref[0, 0] = jnp.concatenate([res[:, 0 * ow:1 * ow] * df, zpad], 1)
        p1_ref[0, 0] = jnp.concatenate([res[:, 1 * ow:2 * ow], zpad], 1)
        p2_ref[1, 0] = jnp.concatenate([res[:, 3 * ow:4 * ow] * db, zpad], 1)
        p1_ref[1, 0] = jnp.concatenate([res[:, 4 * ow:5 * ow], zpad], 1)
        p0_ref[0] = jnp.concatenate([p0full, zpad], 1)
        u_ref[0] = u

    return pl.pallas_call(
        body,
        grid=(BATCH, _NB),
        in_specs=[
            pl.BlockSpec((1, _BLK, 2 * HID), lambda b, i: (b, i, 0)),
            pl.BlockSpec((1, _BLK, d_in), lambda b, i: (b, i, 0)),
            pl.BlockSpec((1, _BLK, HID), lambda b, i: (b, i, 0)),
            pl.BlockSpec((d, 6 * ow), lambda b, i: (0, 0)),
            pl.BlockSpec((1, ow), lambda b, i: (0, 0)),
            pl.BlockSpec((_BLK, 16), lambda b, i: (i, 0)),
            pl.BlockSpec((_BLK, 16), lambda b, i: (i, 0)),
        ],
        out_specs=[
            pl.BlockSpec((2, 1, _BLK, CW), lambda b, i: (0, b, i, 0)),
            pl.BlockSpec((2, 1, _BLK, CW), lambda b, i: (0, b, i, 0)),
            pl.BlockSpec((1, _BLK, CW), lambda b, i: (b, i, 0)),
            pl.BlockSpec((1, _BLK, ow), lambda b, i: (b, i, 0)),
        ],
        out_shape=[
            jax.ShapeDtypeStruct((2, BATCH, NPAD, CW), jnp.float32),
            jax.ShapeDtypeStruct((2, BATCH, NPAD, CW), jnp.float32),
            jax.ShapeDtypeStruct((BATCH, NPAD, CW), jnp.float32),
            jax.ShapeDtypeStruct((BATCH, N, ow), jnp.float32),
        ],
    )


def _post_body(c_ref, u_ref, h_ref, hn_ref):
    c = jnp.tanh(c_ref[0, :, :HID])
    u = u_ref[0]
    hn_ref[0] = u * h_ref[0] + (1.0 - u) * c


_post = pl.pallas_call(
    _post_body,
    grid=(BATCH, _NB),
    in_specs=[
        pl.BlockSpec((1, _BLK, CW), lambda b, i: (b, i, 0)),
        pl.BlockSpec((1, _BLK, HID), lambda b, i: (b, i, 0)),
        pl.BlockSpec((1, _BLK, HID), lambda b, i: (b, i, 0)),
    ],
    out_specs=pl.BlockSpec((1, _BLK, HID), lambda b, i: (b, i, 0)),
    out_shape=jax.ShapeDtypeStruct((BATCH, N, HID), jnp.float32),
)

_ru_pre = [_make_ru_pre(IN_CH), _make_ru_pre(HID)]
_c_pre = [_make_c_pre(IN_CH), _make_c_pre(HID)]


def _wcat(wf, wb):
    return jnp.concatenate([wf[2], wf[1], wf[0], wb[2], wb[1], wb[0]], axis=1)


def kernel(x_seq, edge_index, edge_weight, Wruf0, Wrub0, bru0, Wcf0, Wcb0,
           bc0, Wruf1, Wrub1, bru1, Wcf1, Wcb1, bc1):
    f32 = jnp.float32
    src = edge_index[0]
    dst = edge_index[1]

    pad_n = EPTP - EPT
    fill = (jnp.arange(pad_n, dtype=jnp.int32) * 97) % N
    fill2 = jnp.broadcast_to(fill, (NS, pad_n))

    def pad_idx(a):
        return jnp.concatenate([a.reshape(NS, EPT), fill2], axis=1)

    srcp = pad_idx(src)
    dstp = pad_idx(dst)
    wp = jnp.concatenate(
        [edge_weight.reshape(NS, EPT), jnp.zeros((NS, pad_n), f32)], axis=1)

    def part(g2, s2):
        bucket = s2 // NQR
        order = jnp.argsort(bucket, axis=1, stable=True)
        g_s = jnp.take_along_axis(g2, order, axis=1)
        s_s = jnp.take_along_axis(s2, order, axis=1)
        w_s = jnp.take_along_axis(wp, order, axis=1)
        cns = jnp.stack(
            [(s2 < k * NQR).sum(axis=1) for k in (1, 2, 3)],
            axis=1).astype(jnp.int32)
        s_reb = s_s % NQR
        return g_s, s_reb, w_s, cns

    gf, sf, wf, nf = part(srcp, dstp)
    gb_, sb_, wb_, nb_ = part(dstp, srcp)
    gidx = jnp.stack([gf, gb_])
    sidx = jnp.stack([sf, sb_])
    wts = jnp.stack([wf, wb_])
    cnts = jnp.pad(jnp.stack([nf, nb_]), ((0, 0), (0, 0), (0, 125)))

    ones_tab = jnp.ones((2, BATCH, NPAD, CW), f32)
    zeros_tab = jnp.zeros((2, BATCH, NPAD, CW), f32)
    ones_deg = jnp.ones((2, NPAD * 16), f32)
    y1_deg, _ = _dconv(gidx, sidx, wts, cnts, ones_deg, ones_tab, zeros_tab,
                       zeros_tab[0])
    degf = y1_deg[1, 0, :, 0:16]
    degb = y1_deg[0, 0, :, 0:16]
    degs = jnp.stack([degf, degb]).reshape(2, NPAD * 16)

    wru = [_wcat(Wruf0, Wrub0), _wcat(Wruf1, Wrub1)]
    wc = [_wcat(Wcf0, Wcb0), _wcat(Wcf1, Wcb1)]
    bru = [bru0.reshape(1, 2 * HID), bru1.reshape(1, 2 * HID)]
    bc = [bc0.reshape(1, HID), bc1.reshape(1, HID)]

    xs = jnp.transpose(x_seq, (2, 0, 1, 3))
    h = [jnp.zeros((BATCH, N, HID), f32) for _ in range(NLAYERS)]

    for t in range(TSTEPS):
        x_t = xs[t]
        for l in range(NLAYERS):
            p2, p1, p0 = _ru_pre[l](x_t, h[l], wru[l], bru[l], degf, degb)
            _, out_ru = _dconv(gidx, sidx, wts, cnts, degs, p2, p1, p0)
            p2c, p1c, p0c, u = _c_pre[l](
                out_ru, x_t, h[l], wc[l], bc[l], degf, degb)
            _, out_c = _dconv(gidx, sidx, wts, cnts, degs, p2c, p1c, p0c)
            h[l] = _post(out_c, u, h[l])
            x_t = h[l]

    return (h[0], h[1])


# speedup vs baseline: 6.0849x; 1.1853x over previous
import jax
import jax.numpy as jnp
from jax import lax
from jax.experimental import pallas as pl
from jax.experimental.pallas import tpu as pltpu
from jax.experimental.pallas import tpu_sc as plsc

N = 10000
E = 160000
IN_CH = 128
HID = 64
NLAYERS = 2
BATCH = 2
TSTEPS = 6

NC = 2
NS = 16
CH = 128
CW = 128
EPT = E // NS
NCHUNK = 80
EPTP = NCHUNK * CH
NPAD = 10240
NROUNDS = 4
NQR = NPAD // NROUNDS
RPQ = NQR // NS

_mesh = plsc.VectorSubcoreMesh(
    core_axis_name="c", subcore_axis_name="s", num_cores=NC, num_subcores=NS)


def _make_dconv():
    def body(gidx, sidx, wts, cnts, degs, p2, p1, p0, y1, out,
             gidx_t, sidx_t, wt_t, cnt_t, gb0, gb1, degbuf, acc,
             sem0, sem1, sem2):
        cid = lax.axis_index("c")
        sid = lax.axis_index("s")
        cc = cid
        iota16 = lax.iota(jnp.int32, 16)

        for d in range(2):
            fl = pl.ds(d * EPTP, EPTP)
            pltpu.sync_copy(gidx.at[d, sid], gidx_t.at[fl])
            pltpu.sync_copy(sidx.at[d, sid], sidx_t.at[fl])
            pltpu.sync_copy(wts.at[d, sid], wt_t.at[fl])
            pltpu.sync_copy(cnts.at[d, sid], cnt_t.at[pl.ds(d * 128, 128)])
        cv = [cnt_t[pl.ds(0, 16)], cnt_t[pl.ds(128, 16)]]
        cb = [[0, cv[d][0], cv[d][1], cv[d][2], EPTP] for d in range(2)]

        def scale_scatter(buf, other, d, r, lo, hi):
            def sb(g, c):
                eb = d * EPTP + r * CH + g * 16
                wvec = wt_t[pl.ds(eb, 16)]
                pos = iota16 + (r * CH + g * 16)
                m = (pos >= lo) & (pos < hi)
                wvec = jnp.where(m, wvec, 0.0)
                for lane in range(16):
                    w = wvec[lane]
                    e = g * 16 + lane
                    for kk in range(CW // 16):
                        sl = pl.ds(kk * 16, 16)
                        buf[e, sl] = buf[e, sl] * w
                iv = sidx_t[pl.ds(eb, 16)]
                pltpu.async_copy(buf.at[pl.ds(g * 16, 16)], acc.at[iv],
                                 sem2)
                return c
            lax.fori_loop(0, CH // 16, sb, 0)
            pltpu.make_async_copy(other.at[pl.ds(0, CH)], buf, sem2).wait()

        def sweep(table, d, h):
            lo = cb[d][h]
            hi = cb[d][h + 1]
            t0 = lo // CH
            tend = (hi + CH - 1) // CH
            trips = tend - t0

            def gref(r):
                return table.at[gidx_t.at[pl.ds(d * EPTP + r * CH, CH)]]

            def wait(sem, buf):
                pltpu.make_async_copy(table.at[pl.ds(0, CH)], buf, sem).wait()

            @pl.when(trips > 0)
            def _():
                pltpu.async_copy(gref(t0), gb0, sem0)

            def lbody(jj, c):
                r0 = t0 + 2 * jj
                r1 = r0 + 1
                wait(sem0, gb0)

                @pl.when(r1 < tend)
                def _():
                    pltpu.async_copy(gref(r1), gb1, sem1)

                scale_scatter(gb0, table, d, r0, lo, hi)

                @pl.when(r1 < tend)
                def _():
                    wait(sem1, gb1)

                    @pl.when(r1 + 1 < tend)
                    def _():
                        pltpu.async_copy(gref(r1 + 1), gb0, sem0)

                    scale_scatter(gb1, table, d, r1, lo, hi)
                return c

            lax.fori_loop(0, (trips + 1) // 2, lbody, 0)

        def init_round(src_tab, h):
            pltpu.sync_copy(src_tab.at[pl.ds(h * NQR + sid * RPQ, RPQ)],
                            acc.at[pl.ds(sid * RPQ, RPQ)])
            plsc.subcore_barrier()

        def writeout(d, h, dst, scaled):
            if scaled:
                pltpu.sync_copy(
                    degs.at[d, pl.ds((h * NQR + sid * RPQ) * 16, RPQ * 16)],
                    degbuf)
            for q in range(RPQ // 80):
                qa = sid * RPQ + q * 80
                qd = h * NQR + qa
                pltpu.sync_copy(acc.at[pl.ds(qa, 80)], gb0.at[pl.ds(0, 80)])
                if scaled:
                    def srow(r, c):
                        dvec = degbuf[pl.ds((q * 80 + r) * 16, 16)]
                        inv = 1.0 / jnp.maximum(dvec, 1e-10)
                        for kk in range(CW // 16):
                            sl = pl.ds(kk * 16, 16)
                            gb0[r, sl] = gb0[r, sl] * inv
                        return c
                    lax.fori_loop(0, 80, srow, 0)
                pltpu.sync_copy(gb0.at[pl.ds(0, 80)], dst.at[pl.ds(qd, 80)])

        for d in range(2):
            for h in range(NROUNDS):
                init_round(p1.at[d, cc], h)
                sweep(p2.at[d, cc], d, h)
                plsc.subcore_barrier()
                writeout(d, h, y1.at[d, cc], scaled=True)
        plsc.subcore_barrier()
        for h in range(NROUNDS):
            init_round(p0.at[cc], h)
            for d in range(2):
                sweep(y1.at[d, cc], d, h)
            plsc.subcore_barrier()
            writeout(0, h, out.at[cc], scaled=False)

    return pl.kernel(
        body,
        out_type=[jax.ShapeDtypeStruct((2, BATCH, NPAD, CW), jnp.float32),
                  jax.ShapeDtypeStruct((BATCH, NPAD, CW), jnp.float32)],
        mesh=_mesh,
        scratch_types=[
            pltpu.VMEM((2 * EPTP,), jnp.int32),
            pltpu.VMEM((2 * EPTP,), jnp.int32),
            pltpu.VMEM((2 * EPTP,), jnp.float32),
            pltpu.VMEM((256,), jnp.int32),
            pltpu.VMEM((CH, CW), jnp.float32),
            pltpu.VMEM((CH, CW), jnp.float32),
            pltpu.VMEM((RPQ * 16,), jnp.float32),
            pltpu.VMEM_SHARED((NQR, CW), jnp.float32),
            pltpu.SemaphoreType.DMA,
            pltpu.SemaphoreType.DMA,
            pltpu.SemaphoreType.DMA,
        ],
    )


_dconv = _make_dconv()


_BLK = 2000
_NB = N // _BLK


def _dinv(deg_blk):
    return 1.0 / jnp.maximum(deg_blk[:, 0:1], 1e-10)


def _make_ru_pre(d_in):
    ow = 2 * HID
    d = d_in + HID

    def body(x_ref, h_ref, w_ref, b_ref, degf_ref, degb_ref,
             p2_ref, p1_ref, p0_ref):
        xh = jnp.concatenate([x_ref[0], h_ref[0]], axis=-1)
        res = jnp.dot(xh, w_ref[...], preferred_element_type=jnp.float32)
        df = _dinv(degf_ref[...])
        db = _dinv(degb_ref[...])
        p2_ref[0, 0] = res[:, 0 * ow:1 * ow] * df
        p1_ref[0, 0] = res[:, 1 * ow:2 * ow]
        p2_ref[1, 0] = res[:, 3 * ow:4 * ow] * db
        p1_ref[1, 0] = res[:, 4 * ow:5 * ow]
        p0_ref[0] = res[:, 2 * ow:3 * ow] + res[:, 5 * ow:6 * ow] + b_ref[...]

    return pl.pallas_call(
        body,
        grid=(BATCH, _NB),
        in_specs=[
            pl.BlockSpec((1, _BLK, d_in), lambda b, i: (b, i, 0)),
            pl.BlockSpec((1, _BLK, HID), lambda b, i: (b, i, 0)),
            pl.BlockSpec((d, 6 * ow), lambda b, i: (0, 0)),
            pl.BlockSpec((1, ow), lambda b, i: (0, 0)),
            pl.BlockSpec((_BLK, 16), lambda b, i: (i, 0)),
            pl.BlockSpec((_BLK, 16), lambda b, i: (i, 0)),
        ],
        out_specs=[
            pl.BlockSpec((2, 1, _BLK, CW), lambda b, i: (0, b, i, 0)),
            pl.BlockSpec((2, 1, _BLK, CW), lambda b, i: (0, b, i, 0)),
            pl.BlockSpec((1, _BLK, CW), lambda b, i: (b, i, 0)),
        ],
        out_shape=[
            jax.ShapeDtypeStruct((2, BATCH, NPAD, CW), jnp.float32),
            jax.ShapeDtypeStruct((2, BATCH, NPAD, CW), jnp.float32),
            jax.ShapeDtypeStruct((BATCH, NPAD, CW), jnp.float32),
        ],
    )


def _make_c_pre(d_in):
    ow = HID
    d = d_in + HID

    def body(ru_ref, x_ref, h_ref, w_ref, b_ref, degf_ref, degb_ref,
             p2_ref, p1_ref, p0_ref, u_ref):
        ruv = jax.nn.sigmoid(ru_ref[0])
        r = ruv[:, :HID]
        u = ruv[:, HID:]
        rh = r * h_ref[0]
        xc = jnp.concatenate([x_ref[0], rh], axis=-1)
        res = jnp.dot(xc, w_ref[...], preferred_element_type=jnp.float32)
        df = _dinv(degf_ref[...])
        db = _dinv(degb_ref[...])
        zpad = jnp.zeros((x_ref.shape[1], CW - ow), jnp.float32)
        p0full = res[:, 2 * ow:3 * ow] + res[:, 5 * ow:6 * ow] + b_ref[...]
        p2_ref[0, 0] = jnp.concatenate([res[:, 0 * ow:1 * ow] * df, zpad], 1)
        p1_ref[0, 0] = jnp.concatenate([res[:, 1 * ow:2 * ow], zpad], 1)
        p2_ref[1, 0] = jnp.concatenate([res[:, 3 * ow:4 * ow] * db, zpad], 1)
        p1_ref[1, 0] = jnp.concatenate([res[:, 4 * ow:5 * ow], zpad], 1)
        p0_ref[0] = jnp.concatenate([p0full, zpad], 1)
        u_ref[0] = u

    return pl.pallas_call(
        body,
        grid=(BATCH, _NB),
        in_specs=[
            pl.BlockSpec((1, _BLK, 2 * HID), lambda b, i: (b, i, 0)),
            pl.BlockSpec((1, _BLK, d_in), lambda b, i: (b, i, 0)),
            pl.BlockSpec((1, _BLK, HID), lambda b, i: (b, i, 0)),
            pl.BlockSpec((d, 6 * ow), lambda b, i: (0, 0)),
            pl.BlockSpec((1, ow), lambda b, i: (0, 0)),
            pl.BlockSpec((_BLK, 16), lambda b, i: (i, 0)),
            pl.BlockSpec((_BLK, 16), lambda b, i: (i, 0)),
        ],
        out_specs=[
            pl.BlockSpec((2, 1, _BLK, CW), lambda b, i: (0, b, i, 0)),
            pl.BlockSpec((2, 1, _BLK, CW), lambda b, i: (0, b, i, 0)),
            pl.BlockSpec((1, _BLK, CW), lambda b, i: (b, i, 0)),
            pl.BlockSpec((1, _BLK, ow), lambda b, i: (b, i, 0)),
        ],
        out_shape=[
            jax.ShapeDtypeStruct((2, BATCH, NPAD, CW), jnp.float32),
            jax.ShapeDtypeStruct((2, BATCH, NPAD, CW), jnp.float32),
            jax.ShapeDtypeStruct((BATCH, NPAD, CW), jnp.float32),
            jax.ShapeDtypeStruct((BATCH, N, ow), jnp.float32),
        ],
    )


def _post_body(c_ref, u_ref, h_ref, hn_ref):
    c = jnp.tanh(c_ref[0, :, :HID])
    u = u_ref[0]
    hn_ref[0] = u * h_ref[0] + (1.0 - u) * c


_post = pl.pallas_call(
    _post_body,
    grid=(BATCH, _NB),
    in_specs=[
        pl.BlockSpec((1, _BLK, CW), lambda b, i: (b, i, 0)),
        pl.BlockSpec((1, _BLK, HID), lambda b, i: (b, i, 0)),
        pl.BlockSpec((1, _BLK, HID), lambda b, i: (b, i, 0)),
    ],
    out_specs=pl.BlockSpec((1, _BLK, HID), lambda b, i: (b, i, 0)),
    out_shape=jax.ShapeDtypeStruct((BATCH, N, HID), jnp.float32),
)

_ru_pre = [_make_ru_pre(IN_CH), _make_ru_pre(HID)]
_c_pre = [_make_c_pre(IN_CH), _make_c_pre(HID)]


def _wcat(wf, wb):
    return jnp.concatenate([wf[2], wf[1], wf[0], wb[2], wb[1], wb[0]], axis=1)


def kernel(x_seq, edge_index, edge_weight, Wruf0, Wrub0, bru0, Wcf0, Wcb0,
           bc0, Wruf1, Wrub1, bru1, Wcf1, Wcb1, bc1):
    f32 = jnp.float32
    src = edge_index[0]
    dst = edge_index[1]

    pad_n = EPTP - EPT
    fill = (jnp.arange(pad_n, dtype=jnp.int32) * 97) % N
    fill2 = jnp.broadcast_to(fill, (NS, pad_n))

    def pad_idx(a):
        return jnp.concatenate([a.reshape(NS, EPT), fill2], axis=1)

    srcp = pad_idx(src)
    dstp = pad_idx(dst)
    wp = jnp.concatenate(
        [edge_weight.reshape(NS, EPT), jnp.zeros((NS, pad_n), f32)], axis=1)

    def part(g2, s2):
        bucket = s2 // NQR
        order = jnp.argsort(bucket, axis=1, stable=True)
        g_s = jnp.take_along_axis(g2, order, axis=1)
        s_s = jnp.take_along_axis(s2, order, axis=1)
        w_s = jnp.take_along_axis(wp, order, axis=1)
        cns = jnp.stack(
            [(s2 < k * NQR).sum(axis=1) for k in (1, 2, 3)],
            axis=1).astype(jnp.int32)
        s_reb = s_s % NQR
        return g_s, s_reb, w_s, cns

    gf, sf, wf, nf = part(srcp, dstp)
    gb_, sb_, wb_, nb_ = part(dstp, srcp)
    gidx = jnp.stack([gf, gb_])
    sidx = jnp.stack([sf, sb_])
    wts = jnp.stack([wf, wb_])
    cnts = jnp.pad(jnp.stack([nf, nb_]), ((0, 0), (0, 0), (0, 125)))

    ones_tab = jnp.ones((2, BATCH, NPAD, CW), f32)
    zeros_tab = jnp.zeros((2, BATCH, NPAD, CW), f32)
    ones_deg = jnp.ones((2, NPAD * 16), f32)
    y1_deg, _ = _dconv(gidx, sidx, wts, cnts, ones_deg, ones_tab, zeros_tab,
                       zeros_tab[0])
    degf = y1_deg[1, 0, :, 0:16]
    degb = y1_deg[0, 0, :, 0:16]
    degs = jnp.stack([degf, degb]).reshape(2, NPAD * 16)

    wru = [_wcat(Wruf0, Wrub0), _wcat(Wruf1, Wrub1)]
    wc = [_wcat(Wcf0, Wcb0), _wcat(Wcf1, Wcb1)]
    bru = [bru0.reshape(1, 2 * HID), bru1.reshape(1, 2 * HID)]
    bc = [bc0.reshape(1, HID), bc1.reshape(1, HID)]

    xs = jnp.transpose(x_seq, (2, 0, 1, 3))
    h = [jnp.zeros((BATCH, N, HID), f32) for _ in range(NLAYERS)]

    for t in range(TSTEPS):
        x_t = xs[t]
        for l in range(NLAYERS):
            p2, p1, p0 = _ru_pre[l](x_t, h[l], wru[l], bru[l], degf, degb)
            _, out_ru = _dconv(gidx, sidx, wts, cnts, degs, p2, p1, p0)
            p2c, p1c, p0c, u = _c_pre[l](
                out_ru, x_t, h[l], wc[l], bc[l], degf, degb)
            _, out_c = _dconv(gidx, sidx, wts, cnts, degs, p2c, p1c, p0c)
            h[l] = _post(out_c, u, h[l])
            x_t = h[l]

    return (h[0], h[1])
